# baseline (speedup 1.0000x reference)
import jax
import jax.numpy as jnp
from jax import lax
from jax.experimental import pallas as pl
from jax.experimental.pallas import tpu as pltpu
from jax.experimental.pallas import tpu_sc as plsc

_ALPHA = 0.05
_TARGET = -0.01
_HUBER_DELTA = 0.01

_N = 1048576
_K = max(1, int(_N * _ALPHA))

_NTILES = 16
_CHUNK = _N // _NTILES
_VECS = _CHUNK // 16
_B1 = 2048
_B2 = 2048
_B3 = 1024
_MASK31 = 0x7FFFFFFF


def _splat(x):
    return jnp.full((16,), x, dtype=jnp.int32)


def _splatf(x):
    return jnp.full((16,), x, dtype=jnp.float32)


def _huber_vec(x):
    a = jnp.abs(x)
    return jnp.where(a <= jnp.float32(_HUBER_DELTA),
                     jnp.float32(0.5) * x * x,
                     jnp.float32(_HUBER_DELTA)
                     * (a - jnp.float32(0.5 * _HUBER_DELTA)))


def _sc_body(x_hbm, out_hbm, keys_v, hist_v, merged_v, recv_v, sums_v,
             sums16_v, out_v, load_sem, sh_hist, sh_merged, sh_sums):
    tid = lax.axis_index("s")
    lanes = lax.iota(jnp.int32, 16)
    fifteen = jnp.full((16,), 15, dtype=jnp.int32)
    lane_off1 = lanes * jnp.int32(_B1)
    lane_off3 = lanes * jnp.int32(_B3)
    ones = jnp.ones((16,), jnp.int32)
    zero16 = jnp.zeros((16,), jnp.int32)
    zf = _splatf(0.0)

    with jax.named_scope("ph_load"):
        cp = pltpu.make_async_copy(x_hbm.at[pl.ds(tid * _CHUNK, _CHUNK)],
                                   keys_v, load_sem)
        cp.start()

    def zero_step(i, _):
        hist_v[0, pl.ds(i * 16, 16)] = zero16
        return 0

    with jax.named_scope("ph_zero"):
        @plsc.parallel_loop(0, (_NTILES * _B1) // 16, unroll=4)
        def _z(i):
            hist_v[pl.ds(i * 16, 16)] = zero16
        cp.wait()

    with jax.named_scope("ph_p1"):
        @plsc.parallel_loop(0, _VECS, unroll=4)
        def _p1(i):
            raw = plsc.bitcast(keys_v[pl.ds(i * 16, 16)], jnp.int32)
            key = raw ^ ((raw >> 31) & jnp.int32(_MASK31))
            keys_v[pl.ds(i * 16, 16)] = plsc.bitcast(key, jnp.float32)
            b = (key >> 21) + jnp.int32(1024)
            plsc.addupdate_scatter(hist_v, [lane_off1 + b], ones)

    def merge_level(nbuckets, krem):
        nslices = nbuckets // 16

        @plsc.parallel_loop(0, nslices, unroll=2)
        def _lm(i):
            acc = jnp.zeros((16,), jnp.int32)
            for t in range(_NTILES):
                acc = acc + hist_v[pl.ds(t * nbuckets + i * 16, 16)]
                hist_v[pl.ds(t * nbuckets + i * 16, 16)] = zero16
            merged_v[pl.ds(i * 16, 16)] = acc

        pltpu.sync_copy(merged_v.at[pl.ds(0, nbuckets)],
                        sh_hist.at[tid, pl.ds(0, nbuckets)])
        plsc.subcore_barrier()

        slice_w = nbuckets // _NTILES
        for t in range(_NTILES):
            pltpu.sync_copy(sh_hist.at[t, pl.ds(tid * slice_w, slice_w)],
                            recv_v.at[pl.ds(t * slice_w, slice_w)])

        @plsc.parallel_loop(0, slice_w // 16, unroll=2)
        def _sm(i):
            acc = jnp.zeros((16,), jnp.int32)
            for t in range(_NTILES):
                acc = acc + recv_v[pl.ds(t * slice_w + i * 16, 16)]
            merged_v[pl.ds(i * 16, 16)] = acc
        pltpu.sync_copy(merged_v.at[pl.ds(0, slice_w)],
                        sh_merged.at[pl.ds(tid * slice_w, slice_w)])
        plsc.subcore_barrier()

        pltpu.sync_copy(sh_merged.at[pl.ds(0, nbuckets)],
                        merged_v.at[pl.ds(0, nbuckets)])
        krem_vec = _splat(krem)

        def scan_step(i, carry):
            c_vec, b_acc, c_acc = carry
            mv = merged_v[pl.ds(i * 16, 16)]
            cum = plsc.cumsum(mv) + c_vec
            lt = (cum < krem_vec).astype(jnp.int32)
            b_acc = b_acc + lt
            c_acc = c_acc + lt * mv
            c_vec = lax.gather(
                cum, fifteen[:, None],
                lax.GatherDimensionNumbers(offset_dims=(),
                                           collapsed_slice_dims=(0,),
                                           start_index_map=(0,)),
                (1,), mode=lax.GatherScatterMode.PROMISE_IN_BOUNDS)
            return (c_vec, b_acc, c_acc)

        _, b_acc, c_acc = lax.fori_loop(
            0, nslices, scan_step, (zero16, zero16, zero16))
        return jnp.sum(b_acc), jnp.sum(c_acc)

    with jax.named_scope("ph_m1"):
        b1, c1 = merge_level(_B1, jnp.int32(_K))
    krem2 = jnp.int32(_K) - c1
    hi11 = b1 - jnp.int32(1024)

    hi11_vec = _splat(hi11)

    with jax.named_scope("ph_p2"):
        @plsc.parallel_loop(0, _VECS, unroll=4)
        def _p2(i):
            key = plsc.bitcast(keys_v[pl.ds(i * 16, 16)], jnp.int32)
            m = (key >> 21) == hi11_vec
            b = (key >> 10) & jnp.int32(2047)
            plsc.addupdate_scatter(hist_v, [lane_off1 + b], ones, mask=m)
    with jax.named_scope("ph_m2"):
        b2, c2 = merge_level(_B2, krem2)
    krem3 = krem2 - c2
    hi22 = hi11 * jnp.int32(2048) + b2

    hi22_vec = _splat(hi22)

    with jax.named_scope("ph_p3"):
        @plsc.parallel_loop(0, _VECS, unroll=4)
        def _p3(i):
            key = plsc.bitcast(keys_v[pl.ds(i * 16, 16)], jnp.int32)
            m = (key >> 10) == hi22_vec
            b = key & jnp.int32(1023)
            plsc.addupdate_scatter(hist_v, [lane_off3 + b], ones, mask=m)
    with jax.named_scope("ph_m3"):
        b3, c3 = merge_level(_B3, krem3)

    t_key = hi22 * jnp.int32(1024) + b3
    c_below = c1 + c2 + c3

    t_vec = _splat(t_key)
    m31 = _splat(_MASK31)

    with jax.named_scope("ph_p4"):
        @plsc.parallel_loop(0, _VECS // 4, carry=(zf, zf, zf, zf), unroll=2)
        def acc4(i, accs):
            outs = []
            for j in range(4):
                key = plsc.bitcast(keys_v[pl.ds((i * 4 + j) * 16, 16)],
                                   jnp.int32)
                bits = key ^ ((key >> 31) & m31)
                x = plsc.bitcast(bits, jnp.float32)
                outs.append(accs[j]
                            + jnp.where(key < t_vec, _huber_vec(x), zf))
            return tuple(outs)
        acc = acc4[0] + acc4[1] + acc4[2] + acc4[3]
    s_tile = jnp.sum(acc)

    sums_v[...] = _splatf(s_tile)
    pltpu.sync_copy(sums_v, sh_sums.at[pl.ds(tid * 16, 16)])
    plsc.subcore_barrier()

    @pl.when(tid == 0)
    def _():
        pltpu.sync_copy(sh_sums, sums16_v)

        def fs_step(t, acc):
            return acc + sums16_v[pl.ds(t * 16, 16)]

        s_tot = lax.fori_loop(0, _NTILES, fs_step, zf, unroll=4)

        tb = t_key ^ ((t_key >> 31) & jnp.int32(_MASK31))
        tx = plsc.bitcast(_splat(tb), jnp.float32)
        hub_t = _huber_vec(tx)
        ties = (_splat(_K) - _splat(c_below)).astype(jnp.float32)
        total = s_tot + ties * hub_t
        cvar = -total / jnp.float32(_K)
        viol = jnp.maximum(_splatf(_TARGET) - cvar, zf) * jnp.float32(5.0)
        idx = lax.iota(jnp.int32, 16)
        out_v[...] = jnp.where(idx == 0, cvar, jnp.where(idx == 1, viol, zf))
        pltpu.sync_copy(out_v, out_hbm)


_sc_kernel = pl.kernel(
    _sc_body,
    out_type=jax.ShapeDtypeStruct((16,), jnp.float32),
    mesh=plsc.VectorSubcoreMesh(core_axis_name="c", subcore_axis_name="s",
                                num_cores=1, num_subcores=_NTILES),
    compiler_params=pltpu.CompilerParams(needs_layout_passes=False),
    scratch_types=[
        pltpu.VMEM((_CHUNK,), jnp.float32),
        pltpu.VMEM((_NTILES * _B1,), jnp.int32),
        pltpu.VMEM((_B1,), jnp.int32),
        pltpu.VMEM((_B1,), jnp.int32),
        pltpu.VMEM((16,), jnp.float32),
        pltpu.VMEM((_NTILES * 16,), jnp.float32),
        pltpu.VMEM((16,), jnp.float32),
        pltpu.SemaphoreType.DMA,
        pltpu.VMEM_SHARED((_NTILES, _B1), jnp.int32),
        pltpu.VMEM_SHARED((_B1,), jnp.int32),
        pltpu.VMEM_SHARED((_NTILES * 16,), jnp.float32),
    ],
)


@jax.jit
def kernel(returns):
    out = _sc_kernel(returns)
    return (out[0], out[1])


# speedup vs baseline: 1.0329x; 1.0329x over previous
import jax
import jax.numpy as jnp
from jax import lax
from jax.experimental import pallas as pl
from jax.experimental.pallas import tpu as pltpu
from jax.experimental.pallas import tpu_sc as plsc

_ALPHA = 0.05
_TARGET = -0.01
_HUBER_DELTA = 0.01

_N = 1048576
_K = max(1, int(_N * _ALPHA))

_NTILES = 16
_CHUNK = _N // _NTILES
_VECS = _CHUNK // 16
_B1 = 2048
_B2 = 2048
_B3 = 1024
_MASK31 = 0x7FFFFFFF


def _splat(x):
    return jnp.full((16,), x, dtype=jnp.int32)


def _splatf(x):
    return jnp.full((16,), x, dtype=jnp.float32)


def _huber_vec(x):
    a = jnp.abs(x)
    return jnp.where(a <= jnp.float32(_HUBER_DELTA),
                     jnp.float32(0.5) * x * x,
                     jnp.float32(_HUBER_DELTA)
                     * (a - jnp.float32(0.5 * _HUBER_DELTA)))


def _sc_body(x_hbm, out_hbm, keys_v, hist_v, merged_v, recv_v, sums_v,
             sums16_v, out_v, load_sem, sh_hist, sh_merged, sh_sums):
    tid = lax.axis_index("s")
    lanes = lax.iota(jnp.int32, 16)
    fifteen = jnp.full((16,), 15, dtype=jnp.int32)
    lane_off1 = lanes * jnp.int32(_B1)
    lane_off3 = lanes * jnp.int32(_B3)
    ones = jnp.ones((16,), jnp.int32)
    zero16 = jnp.zeros((16,), jnp.int32)
    zf = _splatf(0.0)

    with jax.named_scope("ph_load"):
        cp = pltpu.make_async_copy(x_hbm.at[pl.ds(tid * _CHUNK, _CHUNK)],
                                   keys_v, load_sem)
        cp.start()

    def zero_step(i, _):
        hist_v[0, pl.ds(i * 16, 16)] = zero16
        return 0

    with jax.named_scope("ph_zero"):
        @plsc.parallel_loop(0, (_NTILES * _B1) // 16, unroll=8)
        def _z(i):
            hist_v[pl.ds(i * 16, 16)] = zero16
        cp.wait()

    with jax.named_scope("ph_p1"):
        @plsc.parallel_loop(0, _VECS, unroll=8)
        def _p1(i):
            raw = plsc.bitcast(keys_v[pl.ds(i * 16, 16)], jnp.int32)
            key = raw ^ ((raw >> 31) & jnp.int32(_MASK31))
            keys_v[pl.ds(i * 16, 16)] = plsc.bitcast(key, jnp.float32)
            b = (key >> 21) + jnp.int32(1024)
            plsc.addupdate_scatter(hist_v, [lane_off1 + b], ones)

    def merge_level(nbuckets, krem):
        nslices = nbuckets // 16

        @plsc.parallel_loop(0, nslices, unroll=2)
        def _lm(i):
            acc = jnp.zeros((16,), jnp.int32)
            for t in range(_NTILES):
                acc = acc + hist_v[pl.ds(t * nbuckets + i * 16, 16)]
                hist_v[pl.ds(t * nbuckets + i * 16, 16)] = zero16
            merged_v[pl.ds(i * 16, 16)] = acc

        pltpu.sync_copy(merged_v.at[pl.ds(0, nbuckets)],
                        sh_hist.at[tid, pl.ds(0, nbuckets)])
        plsc.subcore_barrier()

        slice_w = nbuckets // _NTILES
        for t in range(_NTILES):
            pltpu.sync_copy(sh_hist.at[t, pl.ds(tid * slice_w, slice_w)],
                            recv_v.at[pl.ds(t * slice_w, slice_w)])

        @plsc.parallel_loop(0, slice_w // 16, unroll=2)
        def _sm(i):
            acc = jnp.zeros((16,), jnp.int32)
            for t in range(_NTILES):
                acc = acc + recv_v[pl.ds(t * slice_w + i * 16, 16)]
            merged_v[pl.ds(i * 16, 16)] = acc
        pltpu.sync_copy(merged_v.at[pl.ds(0, slice_w)],
                        sh_merged.at[pl.ds(tid * slice_w, slice_w)])
        plsc.subcore_barrier()

        pltpu.sync_copy(sh_merged.at[pl.ds(0, nbuckets)],
                        merged_v.at[pl.ds(0, nbuckets)])
        krem_vec = _splat(krem)

        def scan_step(i, carry):
            c_vec, b_acc, c_acc = carry
            mv = merged_v[pl.ds(i * 16, 16)]
            cum = plsc.cumsum(mv) + c_vec
            lt = (cum < krem_vec).astype(jnp.int32)
            b_acc = b_acc + lt
            c_acc = c_acc + lt * mv
            c_vec = lax.gather(
                cum, fifteen[:, None],
                lax.GatherDimensionNumbers(offset_dims=(),
                                           collapsed_slice_dims=(0,),
                                           start_index_map=(0,)),
                (1,), mode=lax.GatherScatterMode.PROMISE_IN_BOUNDS)
            return (c_vec, b_acc, c_acc)

        _, b_acc, c_acc = lax.fori_loop(
            0, nslices, scan_step, (zero16, zero16, zero16))
        return jnp.sum(b_acc), jnp.sum(c_acc)

    with jax.named_scope("ph_m1"):
        b1, c1 = merge_level(_B1, jnp.int32(_K))
    krem2 = jnp.int32(_K) - c1
    hi11 = b1 - jnp.int32(1024)

    hi11_vec = _splat(hi11)

    with jax.named_scope("ph_p2"):
        @plsc.parallel_loop(0, _VECS, unroll=8)
        def _p2(i):
            key = plsc.bitcast(keys_v[pl.ds(i * 16, 16)], jnp.int32)
            m = (key >> 21) == hi11_vec
            b = (key >> 10) & jnp.int32(2047)
            plsc.addupdate_scatter(hist_v, [lane_off1 + b], ones, mask=m)
    with jax.named_scope("ph_m2"):
        b2, c2 = merge_level(_B2, krem2)
    krem3 = krem2 - c2
    hi22 = hi11 * jnp.int32(2048) + b2

    hi22_vec = _splat(hi22)

    with jax.named_scope("ph_p3"):
        @plsc.parallel_loop(0, _VECS, unroll=8)
        def _p3(i):
            key = plsc.bitcast(keys_v[pl.ds(i * 16, 16)], jnp.int32)
            m = (key >> 10) == hi22_vec
            b = key & jnp.int32(1023)
            plsc.addupdate_scatter(hist_v, [lane_off3 + b], ones, mask=m)
    with jax.named_scope("ph_m3"):
        b3, c3 = merge_level(_B3, krem3)

    t_key = hi22 * jnp.int32(1024) + b3
    c_below = c1 + c2 + c3

    t_vec = _splat(t_key)
    m31 = _splat(_MASK31)

    delta_vec = _splatf(_HUBER_DELTA)
    with jax.named_scope("ph_p4"):
        @plsc.parallel_loop(0, _VECS // 4, carry=(zf, zf, zf, zf), unroll=2)
        def acc4(i, accs):
            outs = []
            for j in range(4):
                key = plsc.bitcast(keys_v[pl.ds((i * 4 + j) * 16, 16)],
                                   jnp.int32)
                a = plsc.bitcast((key ^ (key >> 31)) & m31, jnp.float32)
                b = jnp.minimum(a, delta_vec)
                hl = jnp.float32(0.5) * b * b                     + jnp.float32(_HUBER_DELTA) * (a - b)
                outs.append(accs[j] + jnp.where(key < t_vec, hl, zf))
            return tuple(outs)
        acc = acc4[0] + acc4[1] + acc4[2] + acc4[3]
    s_tile = jnp.sum(acc)

    sums_v[...] = _splatf(s_tile)
    pltpu.sync_copy(sums_v, sh_sums.at[pl.ds(tid * 16, 16)])
    plsc.subcore_barrier()

    @pl.when(tid == 0)
    def _():
        pltpu.sync_copy(sh_sums, sums16_v)

        def fs_step(t, acc):
            return acc + sums16_v[pl.ds(t * 16, 16)]

        s_tot = lax.fori_loop(0, _NTILES, fs_step, zf, unroll=4)

        tb = t_key ^ ((t_key >> 31) & jnp.int32(_MASK31))
        tx = plsc.bitcast(_splat(tb), jnp.float32)
        hub_t = _huber_vec(tx)
        ties = (_splat(_K) - _splat(c_below)).astype(jnp.float32)
        total = s_tot + ties * hub_t
        cvar = -total / jnp.float32(_K)
        viol = jnp.maximum(_splatf(_TARGET) - cvar, zf) * jnp.float32(5.0)
        idx = lax.iota(jnp.int32, 16)
        out_v[...] = jnp.where(idx == 0, cvar, jnp.where(idx == 1, viol, zf))
        pltpu.sync_copy(out_v, out_hbm)


_sc_kernel = pl.kernel(
    _sc_body,
    out_type=jax.ShapeDtypeStruct((16,), jnp.float32),
    mesh=plsc.VectorSubcoreMesh(core_axis_name="c", subcore_axis_name="s",
                                num_cores=1, num_subcores=_NTILES),
    compiler_params=pltpu.CompilerParams(needs_layout_passes=False),
    scratch_types=[
        pltpu.VMEM((_CHUNK,), jnp.float32),
        pltpu.VMEM((_NTILES * _B1,), jnp.int32),
        pltpu.VMEM((_B1,), jnp.int32),
        pltpu.VMEM((_B1,), jnp.int32),
        pltpu.VMEM((16,), jnp.float32),
        pltpu.VMEM((_NTILES * 16,), jnp.float32),
        pltpu.VMEM((16,), jnp.float32),
        pltpu.SemaphoreType.DMA,
        pltpu.VMEM_SHARED((_NTILES, _B1), jnp.int32),
        pltpu.VMEM_SHARED((_B1,), jnp.int32),
        pltpu.VMEM_SHARED((_NTILES * 16,), jnp.float32),
    ],
)


@jax.jit
def kernel(returns):
    out = _sc_kernel(returns)
    return (out[0], out[1])
